# Initial kernel scaffold; baseline (speedup 1.0000x reference)
import jax
import jax.numpy as jnp
from jax.experimental import pallas as pl


def kernel(x, edge_index, W1, b1, W2, b2):
    raise NotImplementedError("write your pallas kernel here")


# speedup vs baseline: 6.9259x; 6.9259x over previous
import functools

import jax
import jax.numpy as jnp
from jax import lax
from jax.experimental import pallas as pl
from jax.experimental.pallas import tpu as pltpu
from jax.experimental.pallas import tpu_sc as plsc

N = 10000
E = 160000
DIN = 256
H = 100
DP = 128

NC, NS = 2, 16
NW = NC * NS
EPAD = 163840
EW = EPAD // NW
K = 128
NCH = EW // K
NPAD = 10112
RT = NPAD // NS

_MESH = plsc.VectorSubcoreMesh(
    core_axis_name="c", subcore_axis_name="s", num_cores=NC, num_subcores=NS)


@functools.partial(
    pl.kernel,
    out_type=jax.ShapeDtypeStruct((NC * NPAD,), jnp.float32),
    mesh=_MESH,
    scratch_types=[
        pltpu.VMEM((K,), jnp.int32),
        pltpu.VMEM((K,), jnp.float32),
        pltpu.VMEM((RT,), jnp.float32),
        pltpu.VMEM_SHARED((NPAD,), jnp.float32),
    ],
)
def _deg_kernel(dst_hbm, zeros_hbm, ones_hbm, out_hbm,
                idx_v, ones_v, stage_v, acc):
    c = lax.axis_index("c")
    s = lax.axis_index("s")
    wid = s * NC + c
    pltpu.sync_copy(zeros_hbm.at[pl.ds(s * RT, RT)], stage_v)
    pltpu.sync_copy(stage_v, acc.at[pl.ds(s * RT, RT)])
    pltpu.sync_copy(ones_hbm, ones_v)
    plsc.subcore_barrier()

    def body(i, carry):
        base = wid * EW + i * K
        pltpu.sync_copy(dst_hbm.at[pl.ds(base, K)], idx_v)
        pltpu.sync_copy(ones_v, acc.at[idx_v], add=True)
        return carry

    lax.fori_loop(0, NCH, body, 0)
    plsc.subcore_barrier()
    pltpu.sync_copy(acc.at[pl.ds(s * RT, RT)], stage_v)
    pltpu.sync_copy(stage_v, out_hbm.at[pl.ds(c * NPAD + s * RT, RT)])


@functools.partial(
    pl.kernel,
    out_type=jax.ShapeDtypeStruct((NC, NPAD, DP), jnp.float32),
    mesh=_MESH,
    scratch_types=[
        pltpu.VMEM((K,), jnp.int32),
        pltpu.VMEM((K,), jnp.int32),
        pltpu.VMEM((K, DP), jnp.float32),
        pltpu.VMEM_SHARED((NPAD, DP), jnp.float32),
        pltpu.SemaphoreType.DMA,
    ],
)
def _scatter_kernel(g_hbm, src_hbm, dst_hbm, zeros_hbm, out_hbm,
                    sidx, didx, rows, acc, sem):
    c = lax.axis_index("c")
    s = lax.axis_index("s")
    wid = s * NC + c
    pltpu.sync_copy(zeros_hbm.at[pl.ds(s * RT, RT)], acc.at[pl.ds(s * RT, RT)])
    plsc.subcore_barrier()

    def body(i, carry):
        base = wid * EW + i * K
        pltpu.sync_copy(src_hbm.at[pl.ds(base, K)], sidx)
        pltpu.sync_copy(dst_hbm.at[pl.ds(base, K)], didx)
        pltpu.async_copy(g_hbm.at[sidx], rows, sem).wait()
        pltpu.sync_copy(rows, acc.at[didx], add=True)
        return carry

    lax.fori_loop(0, NCH, body, 0)
    plsc.subcore_barrier()
    pltpu.sync_copy(acc.at[pl.ds(s * RT, RT)], out_hbm.at[c, pl.ds(s * RT, RT)])


_RB = 1000


def _mm1_body(x_ref, w_ref, o_ref):
    o_ref[...] = jnp.dot(x_ref[...], w_ref[...],
                         preferred_element_type=jnp.float32)


def _dis(deg0_ref, deg1_ref):
    deg = deg0_ref[...] + deg1_ref[...] + 1.0
    return lax.rsqrt(deg)


def _scale_body(deg0_ref, deg1_ref, h_ref, o_ref):
    o_ref[...] = h_ref[...] * _dis(deg0_ref, deg1_ref)


def _mid_body(accp_ref, deg0_ref, deg1_ref, g1_ref, b1_ref, w2_ref, o_ref):
    dis = _dis(deg0_ref, deg1_ref)
    t = dis * (accp_ref[0] + accp_ref[1] + g1_ref[...]) + b1_ref[...]
    t = jnp.maximum(t, 0.0)
    o_ref[...] = dis * jnp.dot(t, w2_ref[...],
                               preferred_element_type=jnp.float32)


def _fin_body(accp_ref, deg0_ref, deg1_ref, g2_ref, b2_ref, o_ref):
    dis = _dis(deg0_ref, deg1_ref)
    z = dis * (accp_ref[0] + accp_ref[1] + g2_ref[...]) + b2_ref[...]
    z = jnp.maximum(z, 0.0)
    col = lax.broadcasted_iota(jnp.int32, z.shape, 1)
    valid = col < H
    m = jnp.max(jnp.where(valid, z, -jnp.inf), axis=1, keepdims=True)
    ez = jnp.where(valid, jnp.exp(z - m), 0.0)
    lse = jnp.log(jnp.sum(ez, axis=1, keepdims=True)) + m
    o_ref[...] = (z - lse)[:, :H]


def _row_spec(w):
    return pl.BlockSpec((_RB, w), lambda i: (i, 0))


def _full_spec(shape):
    nd = len(shape)
    return pl.BlockSpec(shape, lambda i, _n=nd: (0,) * _n)


_DEG_SPEC = pl.BlockSpec((_RB, 1), lambda i: (i, 0))
_ACCP_SPEC = pl.BlockSpec((NC, _RB, DP), lambda i: (0, i, 0))

_mm1_call = pl.pallas_call(
    _mm1_body, grid=(N // _RB,),
    in_specs=[_row_spec(DIN), _full_spec((DIN, DP))],
    out_specs=_row_spec(DP),
    out_shape=jax.ShapeDtypeStruct((N, DP), jnp.float32))

_scale_call = pl.pallas_call(
    _scale_body, grid=(N // _RB,),
    in_specs=[_DEG_SPEC, _DEG_SPEC, _row_spec(DP)],
    out_specs=_row_spec(DP),
    out_shape=jax.ShapeDtypeStruct((N, DP), jnp.float32))

_mid_call = pl.pallas_call(
    _mid_body, grid=(N // _RB,),
    in_specs=[_ACCP_SPEC, _DEG_SPEC, _DEG_SPEC, _row_spec(DP),
              _full_spec((1, DP)), _full_spec((DP, DP))],
    out_specs=_row_spec(DP),
    out_shape=jax.ShapeDtypeStruct((N, DP), jnp.float32))

_fin_call = pl.pallas_call(
    _fin_body, grid=(N // _RB,),
    in_specs=[_ACCP_SPEC, _DEG_SPEC, _DEG_SPEC, _row_spec(DP),
              _full_spec((1, DP))],
    out_specs=_row_spec(H),
    out_shape=jax.ShapeDtypeStruct((N, H), jnp.float32))


def kernel(x, edge_index, W1, b1, W2, b2):
    src = edge_index[0].astype(jnp.int32)
    dst = edge_index[1].astype(jnp.int32)
    pad = EPAD - E
    srcp = jnp.concatenate([src, jnp.zeros((pad,), jnp.int32)])
    dstp = jnp.concatenate([dst, jnp.full((pad,), N, jnp.int32)])

    zeros_dp = jnp.zeros((NPAD, DP), jnp.float32)
    zeros_1d = jnp.zeros((NPAD,), jnp.float32)
    ones_1d = jnp.ones((K,), jnp.float32)
    W1p = jnp.zeros((DIN, DP), jnp.float32).at[:, :H].set(W1)
    W2p = jnp.zeros((DP, DP), jnp.float32).at[:H, :H].set(W2)
    b1p = jnp.zeros((1, DP), jnp.float32).at[0, :H].set(b1)
    b2p = jnp.zeros((1, DP), jnp.float32).at[0, :H].set(b2)

    degf = _deg_kernel(dstp, zeros_1d, ones_1d)
    deg0 = degf[:N, None]
    deg1 = degf[NPAD:NPAD + N, None]
    h1 = _mm1_call(x, W1p)
    g1 = _scale_call(deg0, deg1, h1)
    acc1 = _scatter_kernel(g1, srcp, dstp, zeros_dp)
    g2 = _mid_call(acc1, deg0, deg1, g1, b1p, W2p)
    acc2 = _scatter_kernel(g2, srcp, dstp, zeros_dp)
    return _fin_call(acc2, deg0, deg1, g2, b2p)


# speedup vs baseline: 8.6891x; 1.2546x over previous
import functools

import jax
import jax.numpy as jnp
from jax import lax
from jax.experimental import pallas as pl
from jax.experimental.pallas import tpu as pltpu
from jax.experimental.pallas import tpu_sc as plsc

N = 10000
E = 160000
DIN = 256
H = 100
DP = 128

NC, NS = 2, 16
NW = NC * NS
EPAD = 163840
EW = EPAD // NW
K = 128
NCH = EW // K
NPAD = 10112
RT = NPAD // NS

_MESH = plsc.VectorSubcoreMesh(
    core_axis_name="c", subcore_axis_name="s", num_cores=NC, num_subcores=NS)


@functools.partial(
    pl.kernel,
    out_type=jax.ShapeDtypeStruct((NC * NPAD,), jnp.float32),
    mesh=_MESH,
    scratch_types=[
        pltpu.VMEM((K,), jnp.int32),
        pltpu.VMEM((K,), jnp.float32),
        pltpu.VMEM((RT,), jnp.float32),
        pltpu.VMEM_SHARED((NPAD,), jnp.float32),
    ],
)
def _deg_kernel(dst_hbm, zeros_hbm, ones_hbm, out_hbm,
                idx_v, ones_v, stage_v, acc):
    c = lax.axis_index("c")
    s = lax.axis_index("s")
    wid = s * NC + c
    pltpu.sync_copy(zeros_hbm.at[pl.ds(s * RT, RT)], stage_v)
    pltpu.sync_copy(stage_v, acc.at[pl.ds(s * RT, RT)])
    pltpu.sync_copy(ones_hbm, ones_v)
    plsc.subcore_barrier()

    def body(i, carry):
        base = wid * EW + i * K
        pltpu.sync_copy(dst_hbm.at[pl.ds(base, K)], idx_v)
        pltpu.sync_copy(ones_v, acc.at[idx_v], add=True)
        return carry

    lax.fori_loop(0, NCH, body, 0)
    plsc.subcore_barrier()
    pltpu.sync_copy(acc.at[pl.ds(s * RT, RT)], stage_v)
    pltpu.sync_copy(stage_v, out_hbm.at[pl.ds(c * NPAD + s * RT, RT)])


NB = 2


@functools.partial(
    pl.kernel,
    out_type=jax.ShapeDtypeStruct((NC, NPAD, DP), jnp.float32),
    mesh=_MESH,
    scratch_types=[
        pltpu.VMEM((NCH, K), jnp.int32),
        pltpu.VMEM((NCH, K), jnp.int32),
        [pltpu.VMEM((K, DP), jnp.float32)] * NB,
        pltpu.VMEM_SHARED((NPAD, DP), jnp.float32),
        [pltpu.SemaphoreType.DMA] * NB,
        [pltpu.SemaphoreType.DMA] * NB,
    ],
)
def _scatter_kernel(g_hbm, src_hbm, dst_hbm, zeros_hbm, out_hbm,
                    sidx, didx, rows, acc, semg, sems):
    c = lax.axis_index("c")
    s = lax.axis_index("s")
    wid = s * NC + c
    pltpu.sync_copy(src_hbm.at[pl.ds(wid * NCH, NCH)], sidx)
    pltpu.sync_copy(dst_hbm.at[pl.ds(wid * NCH, NCH)], didx)
    pltpu.sync_copy(zeros_hbm.at[pl.ds(s * RT, RT)], acc.at[pl.ds(s * RT, RT)])
    plsc.subcore_barrier()

    def start_g(j, b):
        pltpu.async_copy(g_hbm.at[sidx.at[j]], rows[b], semg[b])

    def wait_g(b):
        pltpu.make_async_copy(g_hbm.at[sidx.at[0]], rows[b], semg[b]).wait()

    def start_s(j, b):
        pltpu.async_copy(rows[b], acc.at[didx.at[j]], sems[b], add=True)

    def wait_s(b):
        pltpu.make_async_copy(rows[b], acc.at[didx.at[0]], sems[b]).wait()

    def outer(t, carry):
        for b in range(NB):
            j = t * NB + b
            pb = (b - 1) % NB

            @pl.when(t > 0)
            def _():
                wait_s(b)
            start_g(j, b)

            if b > 0:
                wait_g(pb)
                start_s(j - 1, pb)
            else:
                @pl.when(t > 0)
                def _():
                    wait_g(pb)
                    start_s(j - 1, pb)
        return carry

    lax.fori_loop(0, NCH // NB, outer, 0)
    wait_g(NB - 1)
    start_s(NCH - 1, NB - 1)
    for b in range(NB):
        wait_s(b)
    plsc.subcore_barrier()
    pltpu.sync_copy(acc.at[pl.ds(s * RT, RT)], out_hbm.at[c, pl.ds(s * RT, RT)])


_RB = 1000


def _mm1_body(x_ref, w_ref, o_ref):
    o_ref[...] = jnp.dot(x_ref[...], w_ref[...],
                         preferred_element_type=jnp.float32)


def _dis(deg0_ref, deg1_ref):
    deg = deg0_ref[...] + deg1_ref[...] + 1.0
    return lax.rsqrt(deg)


def _scale_body(deg0_ref, deg1_ref, h_ref, o_ref):
    o_ref[...] = h_ref[...] * _dis(deg0_ref, deg1_ref)


def _mid_body(accp_ref, deg0_ref, deg1_ref, g1_ref, b1_ref, w2_ref, o_ref):
    dis = _dis(deg0_ref, deg1_ref)
    t = dis * (accp_ref[0] + accp_ref[1] + g1_ref[...]) + b1_ref[...]
    t = jnp.maximum(t, 0.0)
    o_ref[...] = dis * jnp.dot(t, w2_ref[...],
                               preferred_element_type=jnp.float32)


def _fin_body(accp_ref, deg0_ref, deg1_ref, g2_ref, b2_ref, o_ref):
    dis = _dis(deg0_ref, deg1_ref)
    z = dis * (accp_ref[0] + accp_ref[1] + g2_ref[...]) + b2_ref[...]
    z = jnp.maximum(z, 0.0)
    col = lax.broadcasted_iota(jnp.int32, z.shape, 1)
    valid = col < H
    m = jnp.max(jnp.where(valid, z, -jnp.inf), axis=1, keepdims=True)
    ez = jnp.where(valid, jnp.exp(z - m), 0.0)
    lse = jnp.log(jnp.sum(ez, axis=1, keepdims=True)) + m
    o_ref[...] = (z - lse)[:, :H]


def _row_spec(w):
    return pl.BlockSpec((_RB, w), lambda i: (i, 0))


def _full_spec(shape):
    nd = len(shape)
    return pl.BlockSpec(shape, lambda i, _n=nd: (0,) * _n)


_DEG_SPEC = pl.BlockSpec((_RB, 1), lambda i: (i, 0))
_ACCP_SPEC = pl.BlockSpec((NC, _RB, DP), lambda i: (0, i, 0))

_mm1_call = pl.pallas_call(
    _mm1_body, grid=(N // _RB,),
    in_specs=[_row_spec(DIN), _full_spec((DIN, DP))],
    out_specs=_row_spec(DP),
    out_shape=jax.ShapeDtypeStruct((N, DP), jnp.float32))

_scale_call = pl.pallas_call(
    _scale_body, grid=(N // _RB,),
    in_specs=[_DEG_SPEC, _DEG_SPEC, _row_spec(DP)],
    out_specs=_row_spec(DP),
    out_shape=jax.ShapeDtypeStruct((N, DP), jnp.float32))

_mid_call = pl.pallas_call(
    _mid_body, grid=(N // _RB,),
    in_specs=[_ACCP_SPEC, _DEG_SPEC, _DEG_SPEC, _row_spec(DP),
              _full_spec((1, DP)), _full_spec((DP, DP))],
    out_specs=_row_spec(DP),
    out_shape=jax.ShapeDtypeStruct((N, DP), jnp.float32))

_fin_call = pl.pallas_call(
    _fin_body, grid=(N // _RB,),
    in_specs=[_ACCP_SPEC, _DEG_SPEC, _DEG_SPEC, _row_spec(DP),
              _full_spec((1, DP))],
    out_specs=_row_spec(H),
    out_shape=jax.ShapeDtypeStruct((N, H), jnp.float32))


def kernel(x, edge_index, W1, b1, W2, b2):
    src = edge_index[0].astype(jnp.int32)
    dst = edge_index[1].astype(jnp.int32)
    pad = EPAD - E
    srcp = jnp.concatenate([src, jnp.zeros((pad,), jnp.int32)])
    dstp = jnp.concatenate([dst, jnp.full((pad,), N, jnp.int32)])

    zeros_dp = jnp.zeros((NPAD, DP), jnp.float32)
    zeros_1d = jnp.zeros((NPAD,), jnp.float32)
    ones_1d = jnp.ones((K,), jnp.float32)
    W1p = jnp.zeros((DIN, DP), jnp.float32).at[:, :H].set(W1)
    W2p = jnp.zeros((DP, DP), jnp.float32).at[:H, :H].set(W2)
    b1p = jnp.zeros((1, DP), jnp.float32).at[0, :H].set(b1)
    b2p = jnp.zeros((1, DP), jnp.float32).at[0, :H].set(b2)

    src2 = srcp.reshape(NW * NCH, K)
    dst2 = dstp.reshape(NW * NCH, K)

    degf = _deg_kernel(dstp, zeros_1d, ones_1d)
    deg0 = degf[:N, None]
    deg1 = degf[NPAD:NPAD + N, None]
    h1 = _mm1_call(x, W1p)
    g1 = _scale_call(deg0, deg1, h1)
    acc1 = _scatter_kernel(g1, src2, dst2, zeros_dp)
    g2 = _mid_call(acc1, deg0, deg1, g1, b1p, W2p)
    acc2 = _scatter_kernel(g2, src2, dst2, zeros_dp)
    return _fin_call(acc2, deg0, deg1, g2, b2p)


# speedup vs baseline: 8.9686x; 1.0322x over previous
import functools

import jax
import jax.numpy as jnp
from jax import lax
from jax.experimental import pallas as pl
from jax.experimental.pallas import tpu as pltpu
from jax.experimental.pallas import tpu_sc as plsc

N = 10000
E = 160000
DIN = 256
H = 100
DP = 128

NC, NS = 2, 16
NW = NC * NS
EPAD = 163840
EW = EPAD // NW
K = 128
NCH = EW // K
NPAD = 10112
RT = NPAD // NS

_MESH = plsc.VectorSubcoreMesh(
    core_axis_name="c", subcore_axis_name="s", num_cores=NC, num_subcores=NS)


@functools.partial(
    pl.kernel,
    out_type=jax.ShapeDtypeStruct((NC * NPAD,), jnp.float32),
    mesh=_MESH,
    scratch_types=[
        pltpu.VMEM((K,), jnp.int32),
        pltpu.VMEM((K,), jnp.float32),
        pltpu.VMEM((RT,), jnp.float32),
        pltpu.VMEM_SHARED((NPAD,), jnp.float32),
    ],
)
def _deg_kernel(dst_hbm, zeros_hbm, ones_hbm, out_hbm,
                idx_v, ones_v, stage_v, acc):
    c = lax.axis_index("c")
    s = lax.axis_index("s")
    wid = s * NC + c
    pltpu.sync_copy(zeros_hbm.at[pl.ds(s * RT, RT)], stage_v)
    pltpu.sync_copy(stage_v, acc.at[pl.ds(s * RT, RT)])
    pltpu.sync_copy(ones_hbm, ones_v)
    plsc.subcore_barrier()

    def body(i, carry):
        base = wid * EW + i * K
        pltpu.sync_copy(dst_hbm.at[pl.ds(base, K)], idx_v)
        pltpu.sync_copy(ones_v, acc.at[idx_v], add=True)
        return carry

    lax.fori_loop(0, NCH, body, 0)
    plsc.subcore_barrier()
    pltpu.sync_copy(acc.at[pl.ds(s * RT, RT)], stage_v)
    pltpu.sync_copy(stage_v, out_hbm.at[pl.ds(c * NPAD + s * RT, RT)])


NB = 2

NCH0 = 64
NCH1 = 16
C0TOT = NS * NCH0
NCHMAX = max(NCH0, NCH1)
NCHP = C0TOT + (NS - 1) * NCH1 + NCHMAX


@functools.partial(
    pl.kernel,
    out_type=jax.ShapeDtypeStruct((NC, NPAD, DP), jnp.float32),
    mesh=_MESH,
    scratch_types=[
        pltpu.VMEM((NCHMAX, K), jnp.int32),
        pltpu.VMEM((NCHMAX, K), jnp.int32),
        [pltpu.VMEM((K, DP), jnp.float32)] * NB,
        pltpu.VMEM_SHARED((NPAD, DP), jnp.float32),
        [pltpu.SemaphoreType.DMA] * NB,
        [pltpu.SemaphoreType.DMA] * NB,
    ],
)
def _scatter_kernel(g_hbm, src_hbm, dst_hbm, zeros_hbm, out_hbm,
                    sidx, didx, rows, acc, semg, sems):
    c = lax.axis_index("c")
    s = lax.axis_index("s")
    base = jnp.where(c == 0, s * NCH0, C0TOT + s * NCH1)
    base = pl.multiple_of(base, 8)
    nch = jnp.where(c == 0, NCH0, NCH1)
    pltpu.sync_copy(src_hbm.at[pl.ds(base, NCHMAX)], sidx)
    pltpu.sync_copy(dst_hbm.at[pl.ds(base, NCHMAX)], didx)
    pltpu.sync_copy(zeros_hbm.at[pl.ds(s * RT, RT)], acc.at[pl.ds(s * RT, RT)])
    plsc.subcore_barrier()

    def start_g(j, b):
        pltpu.async_copy(g_hbm.at[sidx.at[j]], rows[b], semg[b])

    def wait_g(b):
        pltpu.make_async_copy(g_hbm.at[sidx.at[0]], rows[b], semg[b]).wait()

    def start_s(j, b):
        pltpu.async_copy(rows[b], acc.at[didx.at[j]], sems[b], add=True)

    def wait_s(b):
        pltpu.make_async_copy(rows[b], acc.at[didx.at[0]], sems[b]).wait()

    def outer(t, carry):
        for b in range(NB):
            j = t * NB + b
            pb = (b - 1) % NB

            @pl.when(t > 0)
            def _():
                wait_s(b)
            start_g(j, b)

            if b > 0:
                wait_g(pb)
                start_s(j - 1, pb)
            else:
                @pl.when(t > 0)
                def _():
                    wait_g(pb)
                    start_s(j - 1, pb)
        return carry

    lax.fori_loop(0, nch // NB, outer, 0)
    wait_g(NB - 1)
    start_s(nch - 1, NB - 1)
    for b in range(NB):
        wait_s(b)
    plsc.subcore_barrier()
    pltpu.sync_copy(acc.at[pl.ds(s * RT, RT)], out_hbm.at[c, pl.ds(s * RT, RT)])


_RB = 1000


def _mm1_body(x_ref, w_ref, o_ref):
    o_ref[...] = jnp.dot(x_ref[...], w_ref[...],
                         preferred_element_type=jnp.float32)


def _dis(deg0_ref, deg1_ref):
    deg = deg0_ref[...] + deg1_ref[...] + 1.0
    return lax.rsqrt(deg)


def _scale_body(deg0_ref, deg1_ref, h_ref, o_ref):
    o_ref[...] = h_ref[...] * _dis(deg0_ref, deg1_ref)


def _mid_body(accp_ref, deg0_ref, deg1_ref, g1_ref, b1_ref, w2_ref, o_ref):
    dis = _dis(deg0_ref, deg1_ref)
    t = dis * (accp_ref[0] + accp_ref[1] + g1_ref[...]) + b1_ref[...]
    t = jnp.maximum(t, 0.0)
    o_ref[...] = dis * jnp.dot(t, w2_ref[...],
                               preferred_element_type=jnp.float32)


def _fin_body(accp_ref, deg0_ref, deg1_ref, g2_ref, b2_ref, o_ref):
    dis = _dis(deg0_ref, deg1_ref)
    z = dis * (accp_ref[0] + accp_ref[1] + g2_ref[...]) + b2_ref[...]
    z = jnp.maximum(z, 0.0)
    col = lax.broadcasted_iota(jnp.int32, z.shape, 1)
    valid = col < H
    m = jnp.max(jnp.where(valid, z, -jnp.inf), axis=1, keepdims=True)
    ez = jnp.where(valid, jnp.exp(z - m), 0.0)
    lse = jnp.log(jnp.sum(ez, axis=1, keepdims=True)) + m
    o_ref[...] = (z - lse)[:, :H]


def _row_spec(w):
    return pl.BlockSpec((_RB, w), lambda i: (i, 0))


def _full_spec(shape):
    nd = len(shape)
    return pl.BlockSpec(shape, lambda i, _n=nd: (0,) * _n)


_DEG_SPEC = pl.BlockSpec((_RB, 1), lambda i: (i, 0))
_ACCP_SPEC = pl.BlockSpec((NC, _RB, DP), lambda i: (0, i, 0))

_mm1_call = pl.pallas_call(
    _mm1_body, grid=(N // _RB,),
    in_specs=[_row_spec(DIN), _full_spec((DIN, DP))],
    out_specs=_row_spec(DP),
    out_shape=jax.ShapeDtypeStruct((N, DP), jnp.float32))

_scale_call = pl.pallas_call(
    _scale_body, grid=(N // _RB,),
    in_specs=[_DEG_SPEC, _DEG_SPEC, _row_spec(DP)],
    out_specs=_row_spec(DP),
    out_shape=jax.ShapeDtypeStruct((N, DP), jnp.float32))

_mid_call = pl.pallas_call(
    _mid_body, grid=(N // _RB,),
    in_specs=[_ACCP_SPEC, _DEG_SPEC, _DEG_SPEC, _row_spec(DP),
              _full_spec((1, DP)), _full_spec((DP, DP))],
    out_specs=_row_spec(DP),
    out_shape=jax.ShapeDtypeStruct((N, DP), jnp.float32))

_fin_call = pl.pallas_call(
    _fin_body, grid=(N // _RB,),
    in_specs=[_ACCP_SPEC, _DEG_SPEC, _DEG_SPEC, _row_spec(DP),
              _full_spec((1, DP))],
    out_specs=_row_spec(H),
    out_shape=jax.ShapeDtypeStruct((N, H), jnp.float32))


def kernel(x, edge_index, W1, b1, W2, b2):
    src = edge_index[0].astype(jnp.int32)
    dst = edge_index[1].astype(jnp.int32)
    pad = EPAD - E
    srcp = jnp.concatenate([src, jnp.zeros((pad,), jnp.int32)])
    dstp = jnp.concatenate([dst, jnp.full((pad,), N, jnp.int32)])

    zeros_dp = jnp.zeros((NPAD, DP), jnp.float32)
    zeros_1d = jnp.zeros((NPAD,), jnp.float32)
    ones_1d = jnp.ones((K,), jnp.float32)
    W1p = jnp.zeros((DIN, DP), jnp.float32).at[:, :H].set(W1)
    W2p = jnp.zeros((DP, DP), jnp.float32).at[:H, :H].set(W2)
    b1p = jnp.zeros((1, DP), jnp.float32).at[0, :H].set(b1)
    b2p = jnp.zeros((1, DP), jnp.float32).at[0, :H].set(b2)

    pad_rows = NCHP - EPAD // K
    src2 = jnp.concatenate(
        [srcp.reshape(EPAD // K, K), jnp.zeros((pad_rows, K), jnp.int32)])
    dst2 = jnp.concatenate(
        [dstp.reshape(EPAD // K, K), jnp.full((pad_rows, K), N, jnp.int32)])

    degf = _deg_kernel(dstp, zeros_1d, ones_1d)
    deg0 = degf[:N, None]
    deg1 = degf[NPAD:NPAD + N, None]
    h1 = _mm1_call(x, W1p)
    g1 = _scale_call(deg0, deg1, h1)
    acc1 = _scatter_kernel(g1, src2, dst2, zeros_dp)
    g2 = _mid_call(acc1, deg0, deg1, g1, b1p, W2p)
    acc2 = _scatter_kernel(g2, src2, dst2, zeros_dp)
    return _fin_call(acc2, deg0, deg1, g2, b2p)
